# baseline (speedup 1.0000x reference)
import functools

import jax
import jax.numpy as jnp
from jax import lax
from jax.experimental import pallas as pl

_NUM_SAMPLES = 100
_SIGMA = 0.05
_K_FRAC = 0.01
_CCAP = 256

_INTERPRET = False


@functools.lru_cache(maxsize=2)
def _scaled_noise(b: int, d: int):
    noise = jax.random.normal(
        jax.random.key(1), (b, _NUM_SAMPLES, d), dtype=jnp.float32)
    return noise * jnp.float32(_SIGMA)


def _fkey(v):
    bits = lax.bitcast_convert_type(v, jnp.int32)
    return bits ^ ((bits >> 31) & jnp.int32(0x7FFFFFFF))


def _key_to_float(key):
    return lax.bitcast_convert_type(
        key ^ ((key >> 31) & jnp.int32(0x7FFFFFFF)), jnp.float32)


def _thresh_masks(key, kk, axis):
    dsz = jnp.int32(key.shape[axis])
    neg = key >> 31
    mag = key & jnp.int32(0x7FFFFFFF)
    cnt_pos = dsz + jnp.sum(neg, axis=axis, keepdims=True)
    t_pos = cnt_pos >= kk
    k2 = jnp.where(t_pos, kk, kk - cnt_pos)
    elig = (neg < 0) != t_pos
    em = jnp.where(elig, mag, jnp.int32(-1))

    t_mag = jnp.zeros_like(cnt_pos)
    for i in range(31):
        cand = t_mag | (jnp.int32(1) << (30 - i))
        miss = (em - cand) >> 31
        cnt = dsz + jnp.sum(miss, axis=axis, keepdims=True)
        t_mag = jnp.where(cnt >= k2, cand, t_mag)

    gt = ((neg >= 0) & jnp.logical_not(t_pos)) | (em > t_mag)
    eq = em == t_mag
    tkey = jnp.where(t_pos, t_mag, t_mag | jnp.int32(-0x80000000))
    return gt, eq, tkey


def _cumsum_excl(arr, axis):
    c = arr
    sh = 1
    size = arr.shape[axis]
    while sh < size:
        if axis == 0:
            pad = jnp.zeros((sh, arr.shape[1]), jnp.float32)
            c = c + jnp.concatenate([pad, c[:-sh, :]], axis=0)
        else:
            pad = jnp.zeros((arr.shape[0], sh), jnp.float32)
            c = c + jnp.concatenate([pad, c[:, :-sh]], axis=1)
        sh *= 2
    return c - arr


def _positions(gt, eq, k, axis):
    kf = jnp.float32(k)
    gtf = gt.astype(jnp.float32)
    eqf = eq.astype(jnp.float32)
    cnt_gt = jnp.sum(gtf, axis=axis, keepdims=True)
    r = kf - cnt_gt
    packed = gtf + eqf * 4096.0
    cx = _cumsum_excl(packed, axis)
    ce = jnp.floor(cx * (1.0 / 4096.0))
    cg = cx - ce * 4096.0
    member = gt | (eq & (ce < r))
    pos = cg + jnp.minimum(ce, r)
    return jnp.where(member, pos, -1.0)


def _full_path(k, x_row, nz, out_ref):
    n = nz.shape[0]
    v = nz + x_row
    gt, eq, _ = _thresh_masks(_fkey(v), jnp.int32(k), axis=1)
    a = _positions(gt, eq, k, axis=1)
    inv_n = jnp.float32(1.0 / n)
    for j in range(k):
        out_ref[0, j, :] = jnp.sum(
            (a == jnp.float32(j)).astype(jnp.float32), axis=0) * inv_n


def _body(k, x_ref, xcol_ref, nz_ref, nzt_ref, out_ref):
    n = nz_ref.shape[1]
    d = nz_ref.shape[2]
    ccap = _CCAP
    x_row = x_ref[0]
    nz = nz_ref[0]

    keyx = _fkey(x_row)
    gtx, eqx, _ = _thresh_masks(keyx, jnp.int32(ccap), axis=1)
    cnt_gtx = jnp.sum(gtx.astype(jnp.float32), axis=1, keepdims=True)
    ceq = _cumsum_excl(eqx.astype(jnp.float32), axis=1)
    candm = gtx | (eqx & (ceq < (jnp.float32(ccap) - cnt_gtx)))

    posc = _cumsum_excl(candm.astype(jnp.float32), axis=1)
    posci = posc.astype(jnp.int32)
    jio = lax.broadcasted_iota(jnp.int32, (ccap, d), 0)
    g = ((jio == posci) & candm).astype(jnp.float32)

    vct = (jnp.dot(g, nzt_ref[0], preferred_element_type=jnp.float32,
                   precision=lax.Precision.HIGHEST)
           + jnp.dot(g, xcol_ref[0], preferred_element_type=jnp.float32,
                     precision=lax.Precision.HIGHEST))

    gtc, eqc, tkey = _thresh_masks(_fkey(vct), jnp.int32(k), axis=0)
    at = _positions(gtc, eqc, k, axis=0)

    a = at.T
    w = jnp.concatenate(
        [jnp.sum((a == jnp.float32(j)).astype(jnp.float32),
                 axis=0).reshape(1, ccap)
         for j in range(k)], axis=0)
    out_fast = jnp.dot(w, g, preferred_element_type=jnp.float32,
                       precision=lax.Precision.HIGHEST) * jnp.float32(1.0 / n)

    v_out = jnp.where(candm, -jnp.inf, nz + x_row)
    vmax_out = jnp.max(v_out, axis=1, keepdims=True)
    t20f = _key_to_float(tkey)
    safe = jnp.all(t20f > vmax_out.T)

    @pl.when(safe)
    def _():
        out_ref[0] = out_fast

    @pl.when(jnp.logical_not(safe))
    def _():
        _full_path(k, x_row, nz, out_ref)


def kernel(x, train_mode):
    del train_mode
    b, d = x.shape
    k = int(d * _K_FRAC)
    k = max(1, min(k, d))
    k = min(1000, k)
    nz = _scaled_noise(b, d)
    nzt = jnp.swapaxes(nz, 1, 2)

    return pl.pallas_call(
        functools.partial(_body, k),
        grid=(b,),
        in_specs=[
            pl.BlockSpec((1, 1, d), lambda i: (i, 0, 0)),
            pl.BlockSpec((1, d, 1), lambda i: (i, 0, 0)),
            pl.BlockSpec((1, _NUM_SAMPLES, d), lambda i: (i, 0, 0)),
            pl.BlockSpec((1, d, _NUM_SAMPLES), lambda i: (i, 0, 0)),
        ],
        out_specs=pl.BlockSpec((1, k, d), lambda i: (i, 0, 0)),
        out_shape=jax.ShapeDtypeStruct((b, k, d), jnp.float32),
        interpret=_INTERPRET,
    )(x.reshape(b, 1, d), x.reshape(b, d, 1), nz, nzt)


# speedup vs baseline: 1.0270x; 1.0270x over previous
import functools

import jax
import jax.numpy as jnp
from jax import lax
from jax.experimental import pallas as pl

_NUM_SAMPLES = 100
_SIGMA = 0.05
_K_FRAC = 0.01
_CCAP = 256

_INTERPRET = False


@functools.lru_cache(maxsize=2)
def _scaled_noise(b: int, d: int):
    noise = jax.random.normal(
        jax.random.key(1), (b, _NUM_SAMPLES, d), dtype=jnp.float32)
    nz = noise * jnp.float32(_SIGMA)
    nzt = jnp.swapaxes(nz, 1, 2)
    colmax = jnp.max(nz, axis=1).reshape(b, 1, d)
    return (jax.device_put(nz), jax.device_put(nzt),
            jax.device_put(colmax))


def _fkey(v):
    bits = lax.bitcast_convert_type(v, jnp.int32)
    return bits ^ ((bits >> 31) & jnp.int32(0x7FFFFFFF))


def _key_to_float(key):
    return lax.bitcast_convert_type(
        key ^ ((key >> 31) & jnp.int32(0x7FFFFFFF)), jnp.float32)


def _thresh_masks(key, kk, axis):
    dsz = jnp.int32(key.shape[axis])
    neg = key >> 31
    mag = key & jnp.int32(0x7FFFFFFF)
    cnt_pos = dsz + jnp.sum(neg, axis=axis, keepdims=True)
    t_pos = cnt_pos >= kk
    k2 = jnp.where(t_pos, kk, kk - cnt_pos)
    elig = (neg < 0) != t_pos
    em = jnp.where(elig, mag, jnp.int32(-1))

    t_mag = jnp.zeros_like(cnt_pos)
    for i in range(31):
        cand = t_mag | (jnp.int32(1) << (30 - i))
        miss = (em - cand) >> 31
        cnt = dsz + jnp.sum(miss, axis=axis, keepdims=True)
        t_mag = jnp.where(cnt >= k2, cand, t_mag)

    gt = ((neg >= 0) & jnp.logical_not(t_pos)) | (em > t_mag)
    eq = em == t_mag
    tkey = jnp.where(t_pos, t_mag, t_mag | jnp.int32(-0x80000000))
    return gt, eq, tkey


def _cumsum_excl(arr, axis):
    c = arr
    sh = 1
    size = arr.shape[axis]
    while sh < size:
        if axis == 0:
            pad = jnp.zeros((sh, arr.shape[1]), jnp.float32)
            c = c + jnp.concatenate([pad, c[:-sh, :]], axis=0)
        else:
            pad = jnp.zeros((arr.shape[0], sh), jnp.float32)
            c = c + jnp.concatenate([pad, c[:, :-sh]], axis=1)
        sh *= 2
    return c - arr


def _positions(gt, eq, k, axis):
    kf = jnp.float32(k)
    gtf = gt.astype(jnp.float32)
    eqf = eq.astype(jnp.float32)
    cnt_gt = jnp.sum(gtf, axis=axis, keepdims=True)
    r = kf - cnt_gt
    packed = gtf + eqf * 4096.0
    cx = _cumsum_excl(packed, axis)
    ce = jnp.floor(cx * (1.0 / 4096.0))
    cg = cx - ce * 4096.0
    member = gt | (eq & (ce < r))
    pos = cg + jnp.minimum(ce, r)
    return jnp.where(member, pos, -1.0)


def _full_path(k, x_row, nz, out_ref):
    n = nz.shape[0]
    v = nz + x_row
    gt, eq, _ = _thresh_masks(_fkey(v), jnp.int32(k), axis=1)
    a = _positions(gt, eq, k, axis=1)
    inv_n = jnp.float32(1.0 / n)
    for j in range(k):
        out_ref[0, j, :] = jnp.sum(
            (a == jnp.float32(j)).astype(jnp.float32), axis=0) * inv_n


def _body(k, x_ref, xcol_ref, nz_ref, nzt_ref, colmax_ref, out_ref):
    n = nz_ref.shape[1]
    d = nz_ref.shape[2]
    ccap = _CCAP
    x_row = x_ref[0]
    nz = nz_ref[0]

    keyx = _fkey(x_row)
    gtx, eqx, _ = _thresh_masks(keyx, jnp.int32(ccap), axis=1)
    cnt_gtx = jnp.sum(gtx.astype(jnp.float32), axis=1, keepdims=True)
    ceq = _cumsum_excl(eqx.astype(jnp.float32), axis=1)
    candm = gtx | (eqx & (ceq < (jnp.float32(ccap) - cnt_gtx)))

    posc = _cumsum_excl(candm.astype(jnp.float32), axis=1)
    posci = posc.astype(jnp.int32)
    jio = lax.broadcasted_iota(jnp.int32, (ccap, d), 0)
    g = ((jio == posci) & candm).astype(jnp.float32)

    vct = (jnp.dot(g, nzt_ref[0], preferred_element_type=jnp.float32,
                   precision=lax.Precision.HIGHEST)
           + jnp.dot(g, xcol_ref[0], preferred_element_type=jnp.float32,
                     precision=lax.Precision.HIGHEST))

    gtc, eqc, tkey = _thresh_masks(_fkey(vct), jnp.int32(k), axis=0)
    at = _positions(gtc, eqc, k, axis=0)

    a = at.T
    w = jnp.concatenate(
        [jnp.sum((a == jnp.float32(j)).astype(jnp.float32),
                 axis=0).reshape(1, ccap)
         for j in range(k)], axis=0)
    out_fast = jnp.dot(w, g, preferred_element_type=jnp.float32) \
        * jnp.float32(1.0 / n)

    bound = jnp.where(candm, -jnp.inf, x_row + colmax_ref[0])
    bound_max = jnp.max(bound)
    t20f = _key_to_float(tkey)
    safe = jnp.min(t20f) > bound_max

    @pl.when(safe)
    def _():
        out_ref[0] = out_fast

    @pl.when(jnp.logical_not(safe))
    def _():
        _full_path(k, x_row, nz, out_ref)


def kernel(x, train_mode):
    del train_mode
    b, d = x.shape
    k = int(d * _K_FRAC)
    k = max(1, min(k, d))
    k = min(1000, k)
    nz, nzt, colmax = _scaled_noise(b, d)

    return pl.pallas_call(
        functools.partial(_body, k),
        grid=(b,),
        in_specs=[
            pl.BlockSpec((1, 1, d), lambda i: (i, 0, 0)),
            pl.BlockSpec((1, d, 1), lambda i: (i, 0, 0)),
            pl.BlockSpec((1, _NUM_SAMPLES, d), lambda i: (i, 0, 0)),
            pl.BlockSpec((1, d, _NUM_SAMPLES), lambda i: (i, 0, 0)),
            pl.BlockSpec((1, 1, d), lambda i: (i, 0, 0)),
        ],
        out_specs=pl.BlockSpec((1, k, d), lambda i: (i, 0, 0)),
        out_shape=jax.ShapeDtypeStruct((b, k, d), jnp.float32),
        interpret=_INTERPRET,
    )(x.reshape(b, 1, d), x.reshape(b, d, 1), nz, nzt, colmax)
